# Initial kernel scaffold; baseline (speedup 1.0000x reference)
import jax
import jax.numpy as jnp
from jax.experimental import pallas as pl


def kernel(x, edge_index_p_to_c, candidate_indices, params):
    raise NotImplementedError("write your pallas kernel here")


# speedup vs baseline: 5.1347x; 5.1347x over previous
import functools

import jax
import jax.numpy as jnp
from jax import lax
from jax.experimental import pallas as pl
from jax.experimental.pallas import tpu as pltpu
from jax.experimental.pallas import tpu_sc as plsc

_N = 50000
_E = 800000
_HID = 64
_IN = 19
_NEG = 0.01
_K = 64

_HALF = _N // 2
_ACC = 25600
_ZR = _ACC // 16
_C = 256
_NCH = _E // _C
_CPT = -(-_NCH // 16)
_CG = 16
_OUT_SPLIT = 5
_OR = _HALF // _OUT_SPLIT

_mesh = plsc.VectorSubcoreMesh(core_axis_name="c", subcore_axis_name="s")
_sc_params = pltpu.CompilerParams(use_tc_tiling_on_sc=False)


def _lrelu(v):
    return jnp.where(v >= 0, v, v * _NEG)


_BR = 2000


def _embed_body(x_ref, win, bin_, w0, b0, g0, be0, w1, b1, g1, be1, out_ref):
    h = _lrelu(
        jnp.dot(x_ref[...], win[...], preferred_element_type=jnp.float32)
        + bin_[...]
    )
    for w, b, g, be in ((w0, b0, g0, be0), (w1, b1, g1, be1)):
        h2 = _lrelu(
            jnp.dot(h, w[...], preferred_element_type=jnp.float32) + b[...]
        )
        hs = h + h2
        mu = jnp.mean(hs, axis=-1, keepdims=True)
        var = jnp.mean((hs - mu) ** 2, axis=-1, keepdims=True)
        h = (hs - mu) / jnp.sqrt(var + 1e-5) * g[...] + be[...]
    out_ref[...] = h


def _embed(x, win, bin_, w0, b0, g0, be0, w1, b1, g1, be1):
    full = lambda shape: pl.BlockSpec(shape, lambda i: (0, 0))
    return pl.pallas_call(
        _embed_body,
        grid=(_N // _BR,),
        in_specs=[
            pl.BlockSpec((_BR, _IN), lambda i: (i, 0)),
            full((_IN, _HID)), full((1, _HID)),
            full((_HID, _HID)), full((1, _HID)), full((1, _HID)), full((1, _HID)),
            full((_HID, _HID)), full((1, _HID)), full((1, _HID)), full((1, _HID)),
        ],
        out_specs=pl.BlockSpec((_BR, _HID), lambda i: (i, 0)),
        out_shape=jax.ShapeDtypeStruct((_N, _HID), jnp.float32),
    )(x, win, bin_, w0, b0, g0, be0, w1, b1, g1, be1)


def _scatter_indices(pidx, sidx, base):
    for j in range(_C // 16):
        p = pidx[pl.ds(j * 16, 16)]
        local = p - base
        ok = (local >= 0) & (local < _HALF)
        sidx[pl.ds(j * 16, 16)] = jnp.where(ok, local, -1)


@functools.partial(
    pl.kernel,
    out_type=jax.ShapeDtypeStruct((_N, _HID), jnp.float32),
    mesh=_mesh,
    scratch_types=[
        pltpu.VMEM((_C,), jnp.int32),
        pltpu.VMEM((_C,), jnp.int32),
        pltpu.VMEM((_C,), jnp.int32),
        pltpu.VMEM((_C, _HID), jnp.float32),
        pltpu.VMEM_SHARED((_ACC, _HID), jnp.float32),
        pltpu.SemaphoreType.DMA,
    ],
    compiler_params=_sc_params,
)
def _sc_sums(h_hbm, child_hbm, parent_hbm, zeros_hbm, out_hbm,
             cidx, pidx, sidx, rows, acc, sem):
    c = lax.axis_index("c")
    s = lax.axis_index("s")
    base = c * _HALF

    pltpu.sync_copy(zeros_hbm, acc.at[pl.ds(s * _ZR, _ZR)])
    plsc.subcore_barrier()

    def chunk_body(i, carry):
        chunk = s + 16 * i

        @pl.when(chunk < _NCH)
        def _():
            off = chunk * _C
            pltpu.sync_copy(child_hbm.at[pl.ds(off, _C)], cidx)
            pltpu.sync_copy(parent_hbm.at[pl.ds(off, _C)], pidx)
            gather = pltpu.async_copy(h_hbm.at[cidx], rows, sem)
            _scatter_indices(pidx, sidx, base)
            gather.wait()
            pltpu.sync_copy(
                rows,
                acc.at[plsc.Indices(sidx, ignored_value=-1)],
                add=True,
            )

        return carry

    lax.fori_loop(0, _CPT, chunk_body, 0)
    plsc.subcore_barrier()

    @pl.when(s < _OUT_SPLIT)
    def _():
        pltpu.sync_copy(
            acc.at[pl.ds(s * _OR, _OR)],
            out_hbm.at[pl.ds(base + s * _OR, _OR)],
        )


@functools.partial(
    pl.kernel,
    out_type=jax.ShapeDtypeStruct((_N, _CG), jnp.float32),
    mesh=_mesh,
    scratch_types=[
        pltpu.VMEM((_C,), jnp.int32),
        pltpu.VMEM((_C,), jnp.int32),
        pltpu.VMEM((_C, _CG), jnp.float32),
        pltpu.VMEM_SHARED((_ACC, _CG), jnp.float32),
    ],
    compiler_params=_sc_params,
)
def _sc_counts(parent_hbm, zeros_hbm, out_hbm, pidx, sidx, ones, acc):
    c = lax.axis_index("c")
    s = lax.axis_index("s")
    base = c * _HALF

    def fill_ones(i, carry):
        ones[i, pl.ds(0, _CG)] = jnp.full((_CG,), 1.0, jnp.float32)
        return carry

    lax.fori_loop(0, _C, fill_ones, 0)
    pltpu.sync_copy(zeros_hbm, acc.at[pl.ds(s * _ZR, _ZR)])
    plsc.subcore_barrier()

    def chunk_body(i, carry):
        chunk = s + 16 * i

        @pl.when(chunk < _NCH)
        def _():
            off = chunk * _C
            pltpu.sync_copy(parent_hbm.at[pl.ds(off, _C)], pidx)
            _scatter_indices(pidx, sidx, base)
            pltpu.sync_copy(
                ones,
                acc.at[plsc.Indices(sidx, ignored_value=-1)],
                add=True,
            )

        return carry

    lax.fori_loop(0, _CPT, chunk_body, 0)
    plsc.subcore_barrier()

    @pl.when(s < _OUT_SPLIT)
    def _():
        pltpu.sync_copy(
            acc.at[pl.ds(s * _OR, _OR)],
            out_hbm.at[pl.ds(base + s * _OR, _OR)],
        )


@functools.partial(
    pl.kernel,
    out_type=jax.ShapeDtypeStruct((_K, _HID), jnp.float32),
    mesh=_mesh,
    scratch_types=[
        pltpu.VMEM((_K,), jnp.int32),
        pltpu.VMEM((_K, _HID), jnp.float32),
        pltpu.SemaphoreType.DMA,
    ],
    compiler_params=_sc_params,
)
def _sc_cand(h_hbm, cand_hbm, out_hbm, idx_v, rows_v, sem):
    c = lax.axis_index("c")
    s = lax.axis_index("s")

    @pl.when((c == 0) & (s == 0))
    def _():
        pltpu.sync_copy(cand_hbm, idx_v)
        pltpu.async_copy(h_hbm.at[idx_v], rows_v, sem).wait()
        pltpu.sync_copy(rows_v, out_hbm)


def _update_body(h_ref, sum_ref, cnt_ref, w1, b1, w2, b2, out_ref):
    cnt = cnt_ref[...][:, :1]
    mean = sum_ref[...] / jnp.maximum(cnt, 1.0)
    d1 = _lrelu(
        jnp.dot(mean, w1[...], preferred_element_type=jnp.float32) + b1[...]
    )
    delta = jnp.dot(d1, w2[...], preferred_element_type=jnp.float32) + b2[...]
    out_ref[...] = h_ref[...] + delta * (cnt > 0).astype(jnp.float32)


def _update(h, sums, counts, w1, b1, w2, b2):
    full = lambda shape: pl.BlockSpec(shape, lambda i: (0, 0))
    return pl.pallas_call(
        _update_body,
        grid=(_N // _BR,),
        in_specs=[
            pl.BlockSpec((_BR, _HID), lambda i: (i, 0)),
            pl.BlockSpec((_BR, _HID), lambda i: (i, 0)),
            pl.BlockSpec((_BR, _CG), lambda i: (i, 0)),
            full((_HID, _HID)), full((1, _HID)),
            full((_HID, _HID)), full((1, _HID)),
        ],
        out_specs=pl.BlockSpec((_BR, _HID), lambda i: (i, 0)),
        out_shape=jax.ShapeDtypeStruct((_N, _HID), jnp.float32),
    )(h, sums, counts, w1, b1, w2, b2)


def _head_body(ch_ref, wh1, bh1, wh2, bh2, out_ref):
    d1 = _lrelu(
        jnp.dot(ch_ref[...], wh1[...], preferred_element_type=jnp.float32)
        + bh1[...]
    )
    out_ref[...] = (
        jnp.dot(d1, wh2[...], preferred_element_type=jnp.float32) + bh2[...]
    )


def _head(cand_h, wh1, bh1, wh2, bh2):
    return pl.pallas_call(
        _head_body,
        out_shape=jax.ShapeDtypeStruct((_K, 1), jnp.float32),
    )(cand_h, wh1, bh1, wh2, bh2)


def kernel(x, edge_index_p_to_c, candidate_indices, params):
    parent = edge_index_p_to_c[0]
    child = edge_index_p_to_c[1]
    row = lambda v: v.reshape(1, _HID)

    h = _embed(
        x,
        params["W_in"], row(params["b_in"]),
        params["W_l0"], row(params["b_l0"]), row(params["g0"]), row(params["be0"]),
        params["W_l1"], row(params["b_l1"]), row(params["g1"]), row(params["be1"]),
    )

    zeros_h = jnp.zeros((_ZR, _HID), jnp.float32)
    zeros_c = jnp.zeros((_ZR, _CG), jnp.float32)
    counts = _sc_counts(parent, zeros_c)

    for i in range(3):
        sums = _sc_sums(h, child, parent, zeros_h)
        h = _update(
            h, sums, counts,
            params["sW1_%d" % i], row(params["sb1_%d" % i]),
            params["sW2_%d" % i], row(params["sb2_%d" % i]),
        )

    cand_h = _sc_cand(h, candidate_indices)
    scores = _head(
        cand_h,
        params["Wh1"], row(params["bh1"]),
        params["Wh2"], params["bh2"].reshape(1, 1),
    )
    return scores[:, 0]
